# baseline (speedup 1.0000x reference)
import functools

import jax
import jax.numpy as jnp
from jax import lax
from jax.experimental import pallas as pl
from jax.experimental.pallas import tpu as pltpu
from jax.experimental.pallas import tpu_sc as plsc

N_C = 7000
N_D = 3000
NN = 10000
HID = 128
K_PAIRS = 788
PAIR_PAD = 1024
PPW = PAIR_PAD // 32

_BS = 400
_BSL = 512


def _mm_body(x_ref, w_ref, o_ref):
    o_ref[...] = lax.dot_general(
        x_ref[...], w_ref[...], (((1,), (0,)), ((), ())),
        preferred_element_type=jnp.float32)


def _matmul(x, w):
    return pl.pallas_call(
        _mm_body,
        out_shape=jax.ShapeDtypeStruct((x.shape[0], w.shape[1]), jnp.float32),
    )(x, w)


def _deg_body(a_ref, deg_ref):
    s = jnp.sum(a_ref[...], axis=0, keepdims=True)

    @pl.when(pl.program_id(0) == 0)
    def _():
        deg_ref[...] = s

    @pl.when(pl.program_id(0) != 0)
    def _():
        deg_ref[...] += s


def _deg_pass(adj):
    return pl.pallas_call(
        _deg_body,
        grid=(NN // _BS,),
        in_specs=[pl.BlockSpec((_BS, NN), lambda r: (r, 0))],
        out_specs=pl.BlockSpec((1, NN), lambda r: (0, 0)),
        out_shape=jax.ShapeDtypeStruct((1, NN), jnp.float32),
    )(adj)


def _agg_body(degs_ref, h_ref, a_ref, o_ref):
    sb = pl.program_id(0)
    inv_s = lax.rsqrt(degs_ref[...] + 1.0)
    rid = lax.broadcasted_iota(jnp.int32, (_BSL, 1), 0) + sb * _BSL
    ok = rid < NN
    x = jnp.where(ok, h_ref[...] * inv_s, 0.0)
    x_hi = x.astype(jnp.bfloat16)
    x_lo = (x - x_hi.astype(jnp.float32)).astype(jnp.bfloat16)
    x2 = jnp.concatenate([x_hi, x_lo], axis=1)
    a = jnp.where(ok, a_ref[...], 0.0).astype(jnp.bfloat16)
    t = lax.dot_general(a, x2, (((0,), (0,)), ((), ())),
                        preferred_element_type=jnp.float32)
    s = t[:, :HID] + t[:, HID:]

    @pl.when(sb == 0)
    def _():
        o_ref[...] = s

    @pl.when(sb != 0)
    def _():
        o_ref[...] += s


def _gcn_agg(adj, deg_col, h):
    return pl.pallas_call(
        _agg_body,
        grid=(pl.cdiv(NN, _BSL),),
        in_specs=[
            pl.BlockSpec((_BSL, 1), lambda r: (r, 0)),
            pl.BlockSpec((_BSL, HID), lambda r: (r, 0)),
            pl.BlockSpec((_BSL, NN), lambda r: (r, 0)),
        ],
        out_specs=pl.BlockSpec((NN, HID), lambda r: (0, 0)),
        out_shape=jax.ShapeDtypeStruct((NN, HID), jnp.float32),
    )(deg_col, h, adj)


def _xform_body(deg_ref, g_ref, w_ref, o_ref):
    inv_d = lax.rsqrt(deg_ref[...] + 1.0)
    agg = g_ref[...] * inv_d
    o_ref[...] = jnp.maximum(
        lax.dot_general(agg, w_ref[...], (((1,), (0,)), ((), ())),
                        preferred_element_type=jnp.float32), 0.0)


def _xform(deg_col, agg, w):
    return pl.pallas_call(
        _xform_body,
        out_shape=jax.ShapeDtypeStruct((NN, HID), jnp.float32),
    )(deg_col, agg, w)


def _cnn_body(k_ref, b_ref, x0_ref, h1_ref, h2_ref, o_ref):
    x0 = x0_ref[...]
    h1 = h1_ref[...]
    h2 = h2_ref[...]
    for o in range(6):
        acc = x0 * k_ref[o, 0] + h1 * k_ref[o, 1] + h2 * k_ref[o, 2] + b_ref[o]
        o_ref[:, o * HID:(o + 1) * HID] = jnp.maximum(acc, 0.0)


def _cnn(conv_k, conv_b, x0, h1, h2):
    bn = 2000
    return pl.pallas_call(
        _cnn_body,
        grid=(NN // bn,),
        in_specs=[
            pl.BlockSpec(memory_space=pltpu.SMEM),
            pl.BlockSpec(memory_space=pltpu.SMEM),
            pl.BlockSpec((bn, HID), lambda r: (r, 0)),
            pl.BlockSpec((bn, HID), lambda r: (r, 0)),
            pl.BlockSpec((bn, HID), lambda r: (r, 0)),
        ],
        out_specs=pl.BlockSpec((bn, 6 * HID), lambda r: (r, 0)),
        out_shape=jax.ShapeDtypeStruct((NN, 6 * HID), jnp.float32),
    )(conv_k, conv_b, x0, h1, h2)


def _mlp_body(f_ref, w1_ref, b1_ref, w2_ref, b2_ref, w3_ref, b3_ref, w4_ref,
              o_ref):
    def dense(h, w_r, b_r):
        z = lax.dot_general(h, w_r[...], (((1,), (0,)), ((), ())),
                            preferred_element_type=jnp.float32) + b_r[...]
        return jnp.where(z >= 0.0, z, 0.01 * z)

    h = dense(f_ref[...], w1_ref, b1_ref)
    h = dense(h, w2_ref, b2_ref)
    h = dense(h, w3_ref, b3_ref)
    z = jnp.sum(h * w4_ref[...], axis=1, keepdims=True)
    o_ref[...] = 1.0 / (1.0 + jnp.exp(-z))


def _mlp(feats, w1, b1, w2, b2, w3, b3, w4row):
    return pl.pallas_call(
        _mlp_body,
        out_shape=jax.ShapeDtypeStruct((feats.shape[0], 1), jnp.float32),
    )(feats, w1, b1.reshape(1, -1), w2, b2.reshape(1, -1), w3,
      b3.reshape(1, -1), w4row)


_SC_PARAMS = pltpu.CompilerParams(needs_layout_passes=False)


def _sc_feats(cnn_tab, ip, jp):
    mesh = plsc.VectorSubcoreMesh(core_axis_name="c", subcore_axis_name="s")

    @functools.partial(
        pl.kernel,
        out_type=(jax.ShapeDtypeStruct((PAIR_PAD, 6 * HID), jnp.float32),
                  jax.ShapeDtypeStruct((PAIR_PAD, 6 * HID), jnp.float32)),
        mesh=mesh,
        compiler_params=_SC_PARAMS,
        scratch_types=[
            pltpu.VMEM((PPW,), jnp.int32),
            pltpu.VMEM((PPW,), jnp.int32),
            pltpu.VMEM((PPW, 6 * HID), jnp.float32),
            pltpu.VMEM((PPW, 6 * HID), jnp.float32),
            pltpu.SemaphoreType.DMA,
        ],
    )
    def body(tab_hbm, ip_hbm, jp_hbm, fi_hbm, fj_hbm,
             ii_v, jj_v, fi_v, fj_v, sem):
        wid = lax.axis_index("s") * 2 + lax.axis_index("c")
        base = wid * PPW
        pltpu.sync_copy(ip_hbm.at[pl.ds(base, PPW)], ii_v)
        pltpu.sync_copy(jp_hbm.at[pl.ds(base, PPW)], jj_v)
        for k in range(PPW // 16):
            sl = pl.ds(k * 16, 16)
            jj_v[sl] = jj_v[sl] + N_C
        pltpu.async_copy(tab_hbm.at[ii_v], fi_v, sem).wait()
        pltpu.async_copy(tab_hbm.at[jj_v], fj_v, sem).wait()
        pltpu.sync_copy(fi_v, fi_hbm.at[pl.ds(base, PPW)])
        pltpu.sync_copy(fj_v, fj_hbm.at[pl.ds(base, PPW)])

    return body(cnn_tab, ip, jp)


_RELW = 7168


def _relpad_body(r_ref, o_ref):
    o_ref[...] = jnp.concatenate(
        [r_ref[...], jnp.zeros((r_ref.shape[0], _RELW - N_C), jnp.float32)],
        axis=1)


def _relpad(rel_t):
    bsp = 200
    return pl.pallas_call(
        _relpad_body,
        grid=(N_D // bsp,),
        in_specs=[pl.BlockSpec((bsp, N_C), lambda r: (r, 0))],
        out_specs=pl.BlockSpec((bsp, _RELW), lambda r: (r, 0)),
        out_shape=jax.ShapeDtypeStruct((N_D, _RELW), jnp.float32),
    )(rel_t)


def _sc_labels(relp, ip, jp):
    mesh = plsc.VectorSubcoreMesh(core_axis_name="c", subcore_axis_name="s")

    @functools.partial(
        pl.kernel,
        out_type=jax.ShapeDtypeStruct((PAIR_PAD,), jnp.float32),
        mesh=mesh,
        compiler_params=_SC_PARAMS,
        scratch_types=[
            pltpu.VMEM((PPW,), jnp.int32),
            pltpu.VMEM((16,), jnp.int32),
            pltpu.VMEM((16,), jnp.int32),
            pltpu.VMEM((16, _RELW), jnp.float32),
            pltpu.VMEM((PPW,), jnp.float32),
            pltpu.SemaphoreType.DMA,
        ],
    )
    def body(rel_hbm, ip_hbm, jp_hbm, lab_hbm,
             ii_v, jj0_v, jj1_v, rows_v, lab_v, sem):
        wid = lax.axis_index("s") * 2 + lax.axis_index("c")
        base = wid * PPW
        pltpu.sync_copy(ip_hbm.at[pl.ds(base, PPW)], ii_v)
        pltpu.sync_copy(jp_hbm.at[pl.ds(base, 16)], jj0_v)
        pltpu.sync_copy(jp_hbm.at[pl.ds(base + 16, 16)], jj1_v)
        for k, jref in ((0, jj0_v), (1, jj1_v)):
            sl = pl.ds(k * 16, 16)
            pltpu.async_copy(rel_hbm.at[jref], rows_v, sem).wait()
            pid = lax.iota(jnp.int32, 16)
            lab_v[sl] = plsc.load_gather(rows_v, [pid, ii_v[sl]])
        pltpu.sync_copy(lab_v, lab_hbm.at[pl.ds(base, PPW)])

    return body(relp, ip, jp)


def kernel(adj, circ_feature, dis_feature, rel_matrix, train_model,
           trainSet_index, W_rna, W_dis, Wg1, Wg2, conv_k, conv_b,
           mW1, mb1, mW2, mb2, mW3, mb3, mW4):
    ip = jnp.pad(trainSet_index[:, 0], (0, PAIR_PAD - K_PAIRS))
    jp = jnp.pad(trainSet_index[:, 1], (0, PAIR_PAD - K_PAIRS))
    labels_pad = _sc_labels(_relpad(rel_matrix.T), ip, jp)
    adj, labels_pad = lax.optimization_barrier((adj, labels_pad))

    circ_f = _matmul(circ_feature, W_rna)
    dis_f = _matmul(dis_feature, W_dis)
    x0 = jnp.concatenate([circ_f, dis_f], axis=0)

    deg_row = _deg_pass(adj)
    deg_col = deg_row.reshape(NN, 1)
    h1 = _xform(deg_col, _gcn_agg(adj, deg_col, x0), Wg1)
    h2 = _xform(deg_col, _gcn_agg(adj, deg_col, h1), Wg2)

    cnn_out = _cnn(conv_k, conv_b, x0, h1, h2)

    fi, fj = _sc_feats(cnn_out, ip, jp)
    feats = jnp.concatenate([fi, fj], axis=1)

    pred_pad = _mlp(feats, mW1, mb1, mW2, mb2, mW3, mb3, mW4.reshape(1, -1))
    return pred_pad[:K_PAIRS], labels_pad[:K_PAIRS]


# speedup vs baseline: 1.0075x; 1.0075x over previous
import functools

import jax
import jax.numpy as jnp
from jax import lax
from jax.experimental import pallas as pl
from jax.experimental.pallas import tpu as pltpu
from jax.experimental.pallas import tpu_sc as plsc

N_C = 7000
N_D = 3000
NN = 10000
HID = 128
K_PAIRS = 788
PAIR_PAD = 1024
PPW = PAIR_PAD // 32

_BS = 400
_BSL = 512


def _mm_body(x_ref, w_ref, o_ref):
    o_ref[...] = lax.dot_general(
        x_ref[...], w_ref[...], (((1,), (0,)), ((), ())),
        preferred_element_type=jnp.float32)


def _matmul(x, w):
    return pl.pallas_call(
        _mm_body,
        out_shape=jax.ShapeDtypeStruct((x.shape[0], w.shape[1]), jnp.float32),
    )(x, w)


def _deg_body(a_ref, deg_ref):
    s = jnp.sum(a_ref[...], axis=0, keepdims=True)

    @pl.when(pl.program_id(0) == 0)
    def _():
        deg_ref[...] = s

    @pl.when(pl.program_id(0) != 0)
    def _():
        deg_ref[...] += s


def _deg_pass(adj):
    return pl.pallas_call(
        _deg_body,
        grid=(NN // _BS,),
        in_specs=[pl.BlockSpec((_BS, NN), lambda r: (r, 0))],
        out_specs=pl.BlockSpec((1, NN), lambda r: (0, 0)),
        out_shape=jax.ShapeDtypeStruct((1, NN), jnp.float32),
    )(adj)


def _agg_body(degs_ref, h_ref, a_ref, o_ref):
    sb = pl.program_id(0)
    inv_s = lax.rsqrt(degs_ref[...] + 1.0)
    rid = lax.broadcasted_iota(jnp.int32, (_BSL, 1), 0) + sb * _BSL
    ok = rid < NN
    x = jnp.where(ok, h_ref[...] * inv_s, 0.0)
    x_hi = x.astype(jnp.bfloat16)
    x_lo = (x - x_hi.astype(jnp.float32)).astype(jnp.bfloat16)
    x2 = jnp.concatenate([x_hi, x_lo], axis=1)
    a = jnp.where(ok, a_ref[...], 0.0).astype(jnp.bfloat16)
    t = lax.dot_general(a, x2, (((0,), (0,)), ((), ())),
                        preferred_element_type=jnp.float32)
    s = t[:, :HID] + t[:, HID:]

    @pl.when(sb == 0)
    def _():
        o_ref[...] = s

    @pl.when(sb != 0)
    def _():
        o_ref[...] += s


def _gcn_agg(adj, deg_col, h):
    return pl.pallas_call(
        _agg_body,
        grid=(pl.cdiv(NN, _BSL),),
        in_specs=[
            pl.BlockSpec((_BSL, 1), lambda r: (r, 0)),
            pl.BlockSpec((_BSL, HID), lambda r: (r, 0)),
            pl.BlockSpec((_BSL, NN), lambda r: (r, 0)),
        ],
        out_specs=pl.BlockSpec((NN, HID), lambda r: (0, 0)),
        out_shape=jax.ShapeDtypeStruct((NN, HID), jnp.float32),
    )(deg_col, h, adj)


def _xform_body(deg_ref, g_ref, w_ref, o_ref):
    inv_d = lax.rsqrt(deg_ref[...] + 1.0)
    agg = g_ref[...] * inv_d
    o_ref[...] = jnp.maximum(
        lax.dot_general(agg, w_ref[...], (((1,), (0,)), ((), ())),
                        preferred_element_type=jnp.float32), 0.0)


def _xform(deg_col, agg, w):
    return pl.pallas_call(
        _xform_body,
        out_shape=jax.ShapeDtypeStruct((NN, HID), jnp.float32),
    )(deg_col, agg, w)


def _cnn_body(k_ref, b_ref, x0_ref, h1_ref, h2_ref, o_ref):
    x0 = x0_ref[...]
    h1 = h1_ref[...]
    h2 = h2_ref[...]
    for o in range(6):
        acc = x0 * k_ref[o, 0] + h1 * k_ref[o, 1] + h2 * k_ref[o, 2] + b_ref[o]
        o_ref[:, o * HID:(o + 1) * HID] = jnp.maximum(acc, 0.0)


def _cnn(conv_k, conv_b, x0, h1, h2):
    bn = 2000
    return pl.pallas_call(
        _cnn_body,
        grid=(NN // bn,),
        in_specs=[
            pl.BlockSpec(memory_space=pltpu.SMEM),
            pl.BlockSpec(memory_space=pltpu.SMEM),
            pl.BlockSpec((bn, HID), lambda r: (r, 0)),
            pl.BlockSpec((bn, HID), lambda r: (r, 0)),
            pl.BlockSpec((bn, HID), lambda r: (r, 0)),
        ],
        out_specs=pl.BlockSpec((bn, 6 * HID), lambda r: (r, 0)),
        out_shape=jax.ShapeDtypeStruct((NN, 6 * HID), jnp.float32),
    )(conv_k, conv_b, x0, h1, h2)


def _mlp_body(f_ref, w1_ref, b1_ref, w2_ref, b2_ref, w3_ref, b3_ref, w4_ref,
              o_ref):
    def dense(h, w_r, b_r):
        z = lax.dot_general(h, w_r[...], (((1,), (0,)), ((), ())),
                            preferred_element_type=jnp.float32) + b_r[...]
        return jnp.where(z >= 0.0, z, 0.01 * z)

    h = dense(f_ref[...], w1_ref, b1_ref)
    h = dense(h, w2_ref, b2_ref)
    h = dense(h, w3_ref, b3_ref)
    z = jnp.sum(h * w4_ref[...], axis=1, keepdims=True)
    o_ref[...] = 1.0 / (1.0 + jnp.exp(-z))


def _mlp(feats, w1, b1, w2, b2, w3, b3, w4row):
    return pl.pallas_call(
        _mlp_body,
        out_shape=jax.ShapeDtypeStruct((feats.shape[0], 1), jnp.float32),
    )(feats, w1, b1.reshape(1, -1), w2, b2.reshape(1, -1), w3,
      b3.reshape(1, -1), w4row)


_SC_PARAMS = pltpu.CompilerParams(needs_layout_passes=False)


def _sc_feats(cnn_tab, ip, jp):
    mesh = plsc.VectorSubcoreMesh(core_axis_name="c", subcore_axis_name="s")

    @functools.partial(
        pl.kernel,
        out_type=(jax.ShapeDtypeStruct((PAIR_PAD, 6 * HID), jnp.float32),
                  jax.ShapeDtypeStruct((PAIR_PAD, 6 * HID), jnp.float32)),
        mesh=mesh,
        compiler_params=_SC_PARAMS,
        scratch_types=[
            pltpu.VMEM((PPW,), jnp.int32),
            pltpu.VMEM((PPW,), jnp.int32),
            pltpu.VMEM((PPW, 6 * HID), jnp.float32),
            pltpu.VMEM((PPW, 6 * HID), jnp.float32),
            pltpu.SemaphoreType.DMA,
        ],
    )
    def body(tab_hbm, ip_hbm, jp_hbm, fi_hbm, fj_hbm,
             ii_v, jj_v, fi_v, fj_v, sem):
        wid = lax.axis_index("s") * 2 + lax.axis_index("c")
        base = wid * PPW
        pltpu.sync_copy(ip_hbm.at[pl.ds(base, PPW)], ii_v)
        pltpu.sync_copy(jp_hbm.at[pl.ds(base, PPW)], jj_v)
        for k in range(PPW // 16):
            sl = pl.ds(k * 16, 16)
            jj_v[sl] = jj_v[sl] + N_C
        pltpu.async_copy(tab_hbm.at[ii_v], fi_v, sem).wait()
        pltpu.async_copy(tab_hbm.at[jj_v], fj_v, sem).wait()
        pltpu.sync_copy(fi_v, fi_hbm.at[pl.ds(base, PPW)])
        pltpu.sync_copy(fj_v, fj_hbm.at[pl.ds(base, PPW)])

    return body(cnn_tab, ip, jp)


_RELW = 7168


def _relpad_body(r_ref, o_ref):
    o_ref[...] = jnp.concatenate(
        [r_ref[...], jnp.zeros((r_ref.shape[0], _RELW - N_C), jnp.float32)],
        axis=1)


def _relpad(rel_t):
    bsp = 200
    return pl.pallas_call(
        _relpad_body,
        grid=(N_D // bsp,),
        in_specs=[pl.BlockSpec((bsp, N_C), lambda r: (r, 0))],
        out_specs=pl.BlockSpec((bsp, _RELW), lambda r: (r, 0)),
        out_shape=jax.ShapeDtypeStruct((N_D, _RELW), jnp.float32),
    )(rel_t)


def _sc_labels(relp, ip, jp):
    mesh = plsc.VectorSubcoreMesh(core_axis_name="c", subcore_axis_name="s")

    @functools.partial(
        pl.kernel,
        out_type=jax.ShapeDtypeStruct((PAIR_PAD,), jnp.float32),
        mesh=mesh,
        compiler_params=_SC_PARAMS,
        scratch_types=[
            pltpu.VMEM((PPW,), jnp.int32),
            pltpu.VMEM((16,), jnp.int32),
            pltpu.VMEM((16,), jnp.int32),
            pltpu.VMEM((16, _RELW), jnp.float32),
            pltpu.VMEM((PPW,), jnp.float32),
            pltpu.SemaphoreType.DMA,
        ],
    )
    def body(rel_hbm, ip_hbm, jp_hbm, lab_hbm,
             ii_v, jj0_v, jj1_v, rows_v, lab_v, sem):
        wid = lax.axis_index("s") * 2 + lax.axis_index("c")
        base = wid * PPW
        pltpu.sync_copy(ip_hbm.at[pl.ds(base, PPW)], ii_v)
        pltpu.sync_copy(jp_hbm.at[pl.ds(base, 16)], jj0_v)
        pltpu.sync_copy(jp_hbm.at[pl.ds(base + 16, 16)], jj1_v)
        for k, jref in ((0, jj0_v), (1, jj1_v)):
            sl = pl.ds(k * 16, 16)
            pltpu.async_copy(rel_hbm.at[jref], rows_v, sem).wait()
            pid = lax.iota(jnp.int32, 16)
            lab_v[sl] = plsc.load_gather(rows_v, [pid, ii_v[sl]])
        pltpu.sync_copy(lab_v, lab_hbm.at[pl.ds(base, PPW)])

    return body(relp, ip, jp)


def kernel(adj, circ_feature, dis_feature, rel_matrix, train_model,
           trainSet_index, W_rna, W_dis, Wg1, Wg2, conv_k, conv_b,
           mW1, mb1, mW2, mb2, mW3, mb3, mW4):
    ip = jnp.pad(trainSet_index[:, 0], (0, PAIR_PAD - K_PAIRS))
    jp = jnp.pad(trainSet_index[:, 1], (0, PAIR_PAD - K_PAIRS))
    labels_pad = _sc_labels(_relpad(rel_matrix.T), ip, jp)

    circ_f = _matmul(circ_feature, W_rna)
    dis_f = _matmul(dis_feature, W_dis)
    x0 = jnp.concatenate([circ_f, dis_f], axis=0)

    deg_row = _deg_pass(adj)
    deg_col = deg_row.reshape(NN, 1)
    h1 = _xform(deg_col, _gcn_agg(adj, deg_col, x0), Wg1)
    h2 = _xform(deg_col, _gcn_agg(adj, deg_col, h1), Wg2)

    cnn_out = _cnn(conv_k, conv_b, x0, h1, h2)

    fi, fj = _sc_feats(cnn_out, ip, jp)
    feats = jnp.concatenate([fi, fj], axis=1)

    pred_pad = _mlp(feats, mW1, mb1, mW2, mb2, mW3, mb3, mW4.reshape(1, -1))
    return pred_pad[:K_PAIRS], labels_pad[:K_PAIRS]
